# initial kernel scaffold (unmeasured)
import jax
import jax.numpy as jnp
from jax import lax
from jax.experimental import pallas as pl
from jax.experimental.pallas import tpu as pltpu


def kernel(
    x,
):
    def body(*refs):
        pass

    out_shape = jax.ShapeDtypeStruct(..., jnp.float32)
    return pl.pallas_call(body, out_shape=out_shape)(...)


# baseline (device time: 23456 ns/iter reference)
import jax
import jax.numpy as jnp
from jax import lax
from jax.experimental import pallas as pl
from jax.experimental.pallas import tpu as pltpu

N_DEV = 4
M_PER = 8192
N = 1024
BLOCK_M = 512
N_BLOCKS = M_PER // BLOCK_M


def _local_reduce_body(x_ref, out_ref):
    pid = pl.program_id(0)
    my_pos = lax.axis_index("i")

    xb = x_ref[:, :]
    bmax = jnp.max(xb, axis=0, keepdims=True)
    rows = lax.broadcasted_iota(jnp.int32, (BLOCK_M, N), 0)
    bidx = jnp.min(
        jnp.where(xb == bmax, rows, BLOCK_M), axis=0, keepdims=True
    )
    base = my_pos * M_PER + pid * BLOCK_M
    gidx = (base + bidx).astype(jnp.float32)

    @pl.when(pid == 0)
    def _():
        out_ref[0:1, :] = bmax
        out_ref[1:2, :] = gidx

    @pl.when(pid > 0)
    def _():
        rv = out_ref[0:1, :]
        ri = out_ref[1:2, :]
        better = bmax > rv
        out_ref[0:1, :] = jnp.where(better, bmax, rv)
        out_ref[1:2, :] = jnp.where(better, gidx, ri)


def _comm_body(p_ref, out_ref, comm_ref, send_sems, recv_sems):
    my_pos = lax.axis_index("i")

    barrier_sem = pltpu.get_barrier_semaphore()
    for d in range(1, N_DEV):
        pl.semaphore_signal(
            barrier_sem,
            inc=1,
            device_id=((my_pos + d) % N_DEV,),
            device_id_type=pl.DeviceIdType.MESH,
        )
    pl.semaphore_wait(barrier_sem, N_DEV - 1)

    comm_ref[pl.ds(my_pos, 1)] = p_ref[:, :].reshape(1, 2, N)

    sends = []
    for d in range(1, N_DEV):
        peer = (my_pos + d) % N_DEV
        rdma = pltpu.make_async_remote_copy(
            src_ref=comm_ref.at[my_pos],
            dst_ref=comm_ref.at[my_pos],
            send_sem=send_sems.at[d],
            recv_sem=recv_sems.at[my_pos],
            device_id=(peer,),
            device_id_type=pl.DeviceIdType.MESH,
        )
        rdma.start()
        sends.append(rdma)

    for d in range(1, N_DEV):
        peer = (my_pos + d) % N_DEV
        recv = pltpu.make_async_remote_copy(
            src_ref=comm_ref.at[peer],
            dst_ref=comm_ref.at[peer],
            send_sem=send_sems.at[d],
            recv_sem=recv_sems.at[peer],
            device_id=(peer,),
            device_id_type=pl.DeviceIdType.MESH,
        )
        recv.wait_recv()
    for rdma in sends:
        rdma.wait_send()

    rv = comm_ref[0, 0:1, :]
    ri = comm_ref[0, 1:2, :]
    for k in range(1, N_DEV):
        v = comm_ref[k, 0:1, :]
        i = comm_ref[k, 1:2, :]
        better = v > rv
        rv = jnp.where(better, v, rv)
        ri = jnp.where(better, i, ri)
    out_ref[0:1, :] = rv
    out_ref[1:2, :] = ri


def kernel(x):
    partial = pl.pallas_call(
        _local_reduce_body,
        grid=(N_BLOCKS,),
        in_specs=[pl.BlockSpec((BLOCK_M, N), lambda i: (i, 0))],
        out_specs=pl.BlockSpec((2, N), lambda i: (0, 0)),
        out_shape=jax.ShapeDtypeStruct((2, N), jnp.float32),
    )(x)

    return pl.pallas_call(
        _comm_body,
        out_shape=jax.ShapeDtypeStruct((2, N), jnp.float32),
        in_specs=[pl.BlockSpec(memory_space=pltpu.VMEM)],
        out_specs=pl.BlockSpec(memory_space=pltpu.VMEM),
        scratch_shapes=[
            pltpu.VMEM((N_DEV, 2, N), jnp.float32),
            pltpu.SemaphoreType.DMA((N_DEV,)),
            pltpu.SemaphoreType.DMA((N_DEV,)),
        ],
        compiler_params=pltpu.CompilerParams(collective_id=0),
    )(partial)


# device time: 17928 ns/iter; 1.3083x vs baseline; 1.3083x over previous
import jax
import jax.numpy as jnp
from jax import lax
from jax.experimental import pallas as pl
from jax.experimental.pallas import tpu as pltpu

N_DEV = 4
M_PER = 8192
N = 1024
BLOCK_M = 2048
N_BLOCKS = M_PER // BLOCK_M


def _body(x_ref, out_ref, acc_ref, comm_ref, send_sems, recv_sems):
    pid = pl.program_id(0)
    my_pos = lax.axis_index("i")
    barrier_sem = pltpu.get_barrier_semaphore()

    @pl.when(pid == 0)
    def _():
        for d in range(1, N_DEV):
            pl.semaphore_signal(
                barrier_sem,
                inc=1,
                device_id=((my_pos + d) % N_DEV,),
                device_id_type=pl.DeviceIdType.MESH,
            )

    xb = x_ref[:, :]
    bmax = jnp.max(xb, axis=0, keepdims=True)
    rows = lax.broadcasted_iota(jnp.int32, (BLOCK_M, N), 0)
    bidx = jnp.min(jnp.where(xb == bmax, rows, BLOCK_M), axis=0, keepdims=True)
    base = my_pos * M_PER + pid * BLOCK_M
    gidx = (base + bidx).astype(jnp.float32)

    @pl.when(pid == 0)
    def _():
        acc_ref[0:1, :] = bmax
        acc_ref[1:2, :] = gidx

    @pl.when(pid > 0)
    def _():
        rv = acc_ref[0:1, :]
        ri = acc_ref[1:2, :]
        better = bmax > rv
        acc_ref[0:1, :] = jnp.where(better, bmax, rv)
        acc_ref[1:2, :] = jnp.where(better, gidx, ri)

    @pl.when(pid == N_BLOCKS - 1)
    def _():
        pl.semaphore_wait(barrier_sem, N_DEV - 1)
        comm_ref[pl.ds(my_pos, 1)] = acc_ref[:, :].reshape(1, 2, N)

        sends = []
        for d in range(1, N_DEV):
            peer = (my_pos + d) % N_DEV
            rdma = pltpu.make_async_remote_copy(
                src_ref=comm_ref.at[my_pos],
                dst_ref=comm_ref.at[my_pos],
                send_sem=send_sems.at[d],
                recv_sem=recv_sems.at[my_pos],
                device_id=(peer,),
                device_id_type=pl.DeviceIdType.MESH,
            )
            rdma.start()
            sends.append(rdma)

        for d in range(1, N_DEV):
            peer = (my_pos + d) % N_DEV
            recv = pltpu.make_async_remote_copy(
                src_ref=comm_ref.at[peer],
                dst_ref=comm_ref.at[peer],
                send_sem=send_sems.at[d],
                recv_sem=recv_sems.at[peer],
                device_id=(peer,),
                device_id_type=pl.DeviceIdType.MESH,
            )
            recv.wait_recv()
        for rdma in sends:
            rdma.wait_send()

        rv = comm_ref[0, 0:1, :]
        ri = comm_ref[0, 1:2, :]
        for k in range(1, N_DEV):
            v = comm_ref[k, 0:1, :]
            i = comm_ref[k, 1:2, :]
            better = v > rv
            rv = jnp.where(better, v, rv)
            ri = jnp.where(better, i, ri)
        out_ref[0:1, :] = rv
        out_ref[1:2, :] = ri


def kernel(x):
    return pl.pallas_call(
        _body,
        grid=(N_BLOCKS,),
        in_specs=[pl.BlockSpec((BLOCK_M, N), lambda i: (i, 0))],
        out_specs=pl.BlockSpec((2, N), lambda i: (0, 0)),
        out_shape=jax.ShapeDtypeStruct((2, N), jnp.float32),
        scratch_shapes=[
            pltpu.VMEM((2, N), jnp.float32),
            pltpu.VMEM((N_DEV, 2, N), jnp.float32),
            pltpu.SemaphoreType.DMA((N_DEV,)),
            pltpu.SemaphoreType.DMA((N_DEV,)),
        ],
        compiler_params=pltpu.CompilerParams(collective_id=0),
    )(x)
